# baseline (speedup 1.0000x reference)
import functools
import math

import jax
import jax.numpy as jnp
from jax import lax
from jax.experimental import pallas as pl
from jax.experimental.pallas import tpu as pltpu
from jax.experimental.pallas import tpu_sc as plsc

F = 64
K = 64
CUTOFF = 10.0
NB = 3
NOUT = 2
N = 50000
E = 800000
EP = 819200
EROWS = EP // 128

BN = 3584
NP = ((N + BN - 1) // BN) * BN
GN = NP // BN
BE = 8192
GEP = EP // BE

_LOG2 = math.log(2.0)

_NSUB = 16
_NCORE = 2
_DW = EP // (2 * _NSUB)
_MROWS = EROWS // _NSUB
_SUP = 16
_NSUP = _MROWS // _SUP
_RB = 4
_GB = 2
_DCH = 6400
_NPS = NP // _NSUB


def _ssp(x):
    return jnp.maximum(x, 0.0) + jnp.log1p(jnp.exp(-jnp.abs(x))) - _LOG2


def _mm(a, w):
    return jnp.dot(a.astype(jnp.bfloat16), w.astype(jnp.bfloat16),
                   preferred_element_type=jnp.float32)


def _residual(x, W1, b1, W2, b2):
    y = _ssp(x)
    y = _ssp(_mm(y, W1) + b1)
    y = _mm(y, W2) + b2
    return x + y


def _dist_body(rx, ry, rz, ii_hbm, jj_hbm, d2_hbm, rc_v, ii_v, jj_v, d2_v):
    wid = lax.axis_index("s") * _NCORE + lax.axis_index("c")
    base = wid * _DW

    for c, r_hbm in enumerate((rx, ry, rz)):
        pltpu.sync_copy(r_hbm, rc_v)

        for t in range(_DW // _DCH):
            pltpu.sync_copy(ii_hbm.at[pl.ds(base + t * _DCH, _DCH)], ii_v)
            pltpu.sync_copy(jj_hbm.at[pl.ds(base + t * _DCH, _DCH)], jj_v)

            def body(k, _):
                sl = pl.ds(k * 16, 16)
                a = plsc.load_gather(rc_v, [ii_v[sl]])
                b = plsc.load_gather(rc_v, [jj_v[sl]])
                d = a - b
                osl = pl.ds(t * _DCH + k * 16, 16)
                if c == 0:
                    d2_v[osl] = d * d
                else:
                    d2_v[osl] += d * d
                return _

            lax.fori_loop(0, _DCH // 16, body, None)

    pltpu.sync_copy(d2_v, d2_hbm.at[pl.ds(base, _DW)])


@functools.partial(
    pl.kernel,
    out_type=jax.ShapeDtypeStruct((EP,), jnp.float32),
    mesh=plsc.VectorSubcoreMesh(core_axis_name="c", subcore_axis_name="s"),
    compiler_params=pltpu.CompilerParams(needs_layout_passes=False,
                                         use_tc_tiling_on_sc=False),
    scratch_types=[
        pltpu.VMEM((N,), jnp.float32),
        pltpu.VMEM((_DCH,), jnp.int32),
        pltpu.VMEM((_DCH,), jnp.int32),
        pltpu.VMEM((_DW,), jnp.float32),
    ],
)
def _sc_dist(rx, ry, rz, ii_hbm, jj_hbm, d2_hbm, rc_v, ii_v, jj_v, d2_v):
    _dist_body(rx, ry, rz, ii_hbm, jj_hbm, d2_hbm, rc_v, ii_v, jj_v, d2_v)


def _make_sc_msg(boff):
    def _msg_body(g_hbm, xj_hbm, ii_hbm, jj_hbm, zero_hbm, msg_hbm,
                  rows, gbuf, ii_sup, jj_sup, acc_sh, sem_in, sem_s):
        c = lax.axis_index("c")
        s = lax.axis_index("s")
        f0 = boff + c * 32

        pltpu.sync_copy(zero_hbm.at[pl.ds(s * _NPS, _NPS)],
                        acc_sh.at[pl.ds(s * _NPS, _NPS)])
        plsc.subcore_barrier()

        def fire(sup, j):
            row = s * _MROWS + sup * _SUP + j
            rb, gb = j % _RB, j % _GB
            pltpu.async_copy(xj_hbm.at[jj_sup.at[j]], rows[rb], sem_in[rb])
            pltpu.async_copy(g_hbm.at[pl.ds(row * 128, 128), pl.ds(f0, 32)],
                             gbuf[gb], sem_in[rb])

        def wait_in(j):
            rb, gb = j % _RB, j % _GB
            pltpu.make_async_copy(xj_hbm.at[jj_sup.at[j]], rows[rb],
                                  sem_in[rb]).wait()
            pltpu.make_async_copy(g_hbm.at[pl.ds(0, 128), pl.ds(f0, 32)],
                                  gbuf[gb], sem_in[rb]).wait()

        def fire_scatter(j):
            rb = j % _RB
            pltpu.async_copy(rows[rb], acc_sh.at[ii_sup.at[j]], sem_s[rb],
                             add=True)

        def drain_scatter(j):
            rb = j % _RB
            pltpu.make_async_copy(rows[rb], acc_sh.at[ii_sup.at[j]],
                                  sem_s[rb]).wait()

        def superblock(sup, _):
            @pl.when(sup > 0)
            def _():
                for j in range(_SUP - _RB, _SUP):
                    drain_scatter(j)

            r0 = s * _MROWS + sup * _SUP
            pltpu.sync_copy(ii_hbm.at[pl.ds(r0, _SUP)], ii_sup)
            pltpu.sync_copy(jj_hbm.at[pl.ds(c * EROWS + r0, _SUP)], jj_sup)

            for j in range(2):
                fire(sup, j)

            for j in range(_SUP):
                wait_in(j)

                def mul(e, _):
                    rb = j % _RB
                    gb = j % _GB
                    rows[rb][e, pl.ds(0, 16)] *= gbuf[gb][e, pl.ds(0, 16)]
                    rows[rb][e, pl.ds(16, 16)] *= gbuf[gb][e, pl.ds(16, 16)]
                    return _

                lax.fori_loop(0, 128, mul, None)
                fire_scatter(j)
                if j + 2 < _SUP:
                    if j - 2 >= 0:
                        drain_scatter(j - 2)
                    fire(sup, j + 2)
            return _

        lax.fori_loop(0, _NSUP, superblock, None)
        for j in range(_SUP - _RB, _SUP):
            drain_scatter(j)

        plsc.subcore_barrier()
        pltpu.sync_copy(acc_sh.at[pl.ds(s * _NPS, _NPS)],
                        msg_hbm.at[pl.ds(c * NP + s * _NPS, _NPS)])

    return functools.partial(
        pl.kernel,
        out_type=jax.ShapeDtypeStruct((2 * NP, 32), jnp.float32),
        mesh=plsc.VectorSubcoreMesh(core_axis_name="c", subcore_axis_name="s"),
        compiler_params=pltpu.CompilerParams(needs_layout_passes=False,
                                             use_tc_tiling_on_sc=False),
        scratch_types=[
            [pltpu.VMEM((128, 32), jnp.float32)] * _RB,
            [pltpu.VMEM((128, 32), jnp.float32)] * _GB,
            pltpu.VMEM((_SUP, 128), jnp.int32),
            pltpu.VMEM((_SUP, 128), jnp.int32),
            pltpu.VMEM_SHARED((NP, 32), jnp.float32),
            [pltpu.SemaphoreType.DMA] * _RB,
            [pltpu.SemaphoreType.DMA] * _RB,
        ],
    )(_msg_body)


_SC_MSG = [_make_sc_msg(b * F) for b in range(NB)]


def _k1_body(f_ref, w0_ref, b0_ref, w1_ref, b1_ref, w2_ref, b2_ref, x_ref):
    x = _ssp(_mm(f_ref[...], w0_ref[...]) + b0_ref[...])
    x = _ssp(_mm(x, w1_ref[...]) + b1_ref[...])
    x_ref[...] = _mm(x, w2_ref[...]) + b2_ref[...]


def _k1(f_pad, p):
    w0 = jnp.pad(p['W_em0'], ((0, 4), (0, 0)))
    return pl.pallas_call(
        _k1_body,
        grid=(GN,),
        in_specs=[
            pl.BlockSpec((BN, 8), lambda i: (i, 0)),
            pl.BlockSpec((8, F), lambda i: (0, 0)),
            pl.BlockSpec((1, F), lambda i: (0, 0)),
            pl.BlockSpec((F, F), lambda i: (0, 0)),
            pl.BlockSpec((1, F), lambda i: (0, 0)),
            pl.BlockSpec((F, F), lambda i: (0, 0)),
            pl.BlockSpec((1, F), lambda i: (0, 0)),
        ],
        out_specs=pl.BlockSpec((BN, F), lambda i: (i, 0)),
        out_shape=jax.ShapeDtypeStruct((NP, F), jnp.float32),
    )(f_pad, w0, p['b_em0'][None], p['W_em1'], p['b_em1'][None],
      p['W_em2'], p['b_em2'][None])


def _kg_body(d2_ref, cen_ref, wid_ref, wg_ref, g_ref):
    i = pl.program_id(0)
    d2 = d2_ref[pl.ds(i, 1), :]
    d = jnp.sqrt(jnp.maximum(d2, 0.0))
    xx = d * (1.0 / CUTOFF)
    x3 = xx * xx * xx
    x4 = x3 * xx
    x5 = x4 * xx
    cf = jnp.where(xx < 1.0, 1.0 - 6.0 * x5 + 15.0 * x4 - 10.0 * x3, 0.0)
    ed = jnp.broadcast_to(jnp.exp(-d), (K, BE))
    t = ed - cen_ref[...]
    rbf = jnp.broadcast_to(cf, (K, BE)) * jnp.exp(-wid_ref[...] * t * t)
    y = lax.dot_general(rbf.astype(jnp.bfloat16), wg_ref[...].astype(jnp.bfloat16),
                        dimension_numbers=(((0,), (0,)), ((), ())),
                        preferred_element_type=jnp.float32)
    g_ref[...] = jnp.concatenate([y, jnp.zeros((BE, 64), jnp.float32)], axis=1)


def _kg(d2, p):
    wg_cat = jnp.concatenate([p['Wg'][b] for b in range(NB)], axis=1)
    return pl.pallas_call(
        _kg_body,
        grid=(GEP,),
        in_specs=[
            pl.BlockSpec((GEP, BE), lambda i: (0, 0)),
            pl.BlockSpec((K, 1), lambda i: (0, 0)),
            pl.BlockSpec((K, 1), lambda i: (0, 0)),
            pl.BlockSpec((K, NB * F), lambda i: (0, 0)),
        ],
        out_specs=pl.BlockSpec((BE, 256), lambda i: (i, 0)),
        out_shape=jax.ShapeDtypeStruct((EP, 256), jnp.float32),
    )(d2, p['centers'][:, None], p['widths'][:, None], wg_cat)


def _k2_body(x_ref, wij_ref, bij_ref, xi_ref, xj_ref):
    xa = _ssp(x_ref[...])
    y = _mm(xa, wij_ref[...]) + bij_ref[...]
    xi_ref[...] = y[:, :F]
    xj_ref[0] = y[:, F:F + 32]
    xj_ref[1] = y[:, F + 32:]


def _k2(x, wij, bij):
    return pl.pallas_call(
        _k2_body,
        grid=(GN,),
        in_specs=[
            pl.BlockSpec((BN, F), lambda i: (i, 0)),
            pl.BlockSpec((F, 2 * F), lambda i: (0, 0)),
            pl.BlockSpec((1, 2 * F), lambda i: (0, 0)),
        ],
        out_specs=[
            pl.BlockSpec((BN, F), lambda i: (i, 0)),
            pl.BlockSpec((2, BN, 32), lambda i: (0, i, 0)),
        ],
        out_shape=[
            jax.ShapeDtypeStruct((NP, F), jnp.float32),
            jax.ShapeDtypeStruct((2, NP, 32), jnp.float32),
        ],
    )(x, wij, bij)


def _k3_body_fused(x_ref, xi_ref, msg_ref,
                   wri1_ref, bri1_ref, wri2_ref, bri2_ref, wm_ref, bm_ref, u_ref,
                   wra1_ref, bra1_ref, wra2_ref, bra2_ref,
                   wro1_ref, bro1_ref, wro2_ref, bro2_ref, wout_ref, bout_ref,
                   wij2_ref, bij2_ref,
                   xn_ref, out_ref, xi2_ref, xj2_ref):
    x = x_ref[...]
    m = xi_ref[...] + jnp.concatenate([msg_ref[0], msg_ref[1]], axis=1)
    for r in range(3):
        m = _residual(m, wri1_ref[r], bri1_ref[r][None], wri2_ref[r], bri2_ref[r][None])
    m = _ssp(m)
    xn = u_ref[...] * x + _mm(m, wm_ref[...]) + bm_ref[...]
    for r in range(2):
        xn = _residual(xn, wra1_ref[r], bra1_ref[r][None], wra2_ref[r], bra2_ref[r][None])
    y = _residual(xn, wro1_ref[...], bro1_ref[...], wro2_ref[...], bro2_ref[...])
    y = _ssp(y)
    out = _mm(y, wout_ref[...]) + bout_ref[...]
    xn_ref[...] = xn
    out_ref[...] = out
    if xi2_ref is not None:
        y2 = _mm(_ssp(xn), wij2_ref[...]) + bij2_ref[...]
        xi2_ref[...] = y2[:, :F]
        xj2_ref[0] = y2[:, F:F + 32]
        xj2_ref[1] = y2[:, F + 32:]


def _k3(x, xi, msg, pb, wij2=None, bij2=None):
    full = lambda shape: pl.BlockSpec(shape, lambda i: tuple(0 for _ in shape))
    fused = wij2 is not None
    in_specs = [
        pl.BlockSpec((BN, F), lambda i: (i, 0)),
        pl.BlockSpec((BN, F), lambda i: (i, 0)),
        pl.BlockSpec((2, BN, 32), lambda i: (0, i, 0)),
        full((3, F, F)), full((3, F)), full((3, F, F)), full((3, F)),
        full((F, F)), full((1, F)), full((1, F)),
        full((2, F, F)), full((2, F)), full((2, F, F)), full((2, F)),
        full((F, F)), full((1, F)), full((F, F)), full((1, F)),
        full((F, NOUT)), full((1, NOUT)),
    ]
    out_specs = [
        pl.BlockSpec((BN, F), lambda i: (i, 0)),
        pl.BlockSpec((BN, NOUT), lambda i: (i, 0)),
    ]
    out_shape = [
        jax.ShapeDtypeStruct((NP, F), jnp.float32),
        jax.ShapeDtypeStruct((NP, NOUT), jnp.float32),
    ]
    args = [x, xi, msg,
            pb['Wri1'], pb['bri1'], pb['Wri2'], pb['bri2'],
            pb['Wm'], pb['bm'][None], pb['u'][None],
            pb['Wra1'], pb['bra1'], pb['Wra2'], pb['bra2'],
            pb['Wro1'], pb['bro1'][None], pb['Wro2'], pb['bro2'][None],
            pb['Wout'], pb['bout'][None]]
    if fused:
        in_specs += [full((F, 2 * F)), full((1, 2 * F))]
        out_specs += [
            pl.BlockSpec((BN, F), lambda i: (i, 0)),
            pl.BlockSpec((2, BN, 32), lambda i: (0, i, 0)),
        ]
        out_shape += [
            jax.ShapeDtypeStruct((NP, F), jnp.float32),
            jax.ShapeDtypeStruct((2, NP, 32), jnp.float32),
        ]
        args += [wij2, bij2]
        body = _k3_body_fused
    else:
        def body(*refs):
            _k3_body_fused(*refs[:20], None, None, *refs[20:], None, None)
    return pl.pallas_call(
        body,
        grid=(GN,),
        in_specs=in_specs,
        out_specs=out_specs,
        out_shape=out_shape,
    )(*args)


def _k4_body(o0_ref, o1_ref, o2_ref, out_ref, nh_ref):
    i = pl.program_id(0)
    o0, o1, o2 = o0_ref[...], o1_ref[...], o2_ref[...]
    out_ref[...] = jnp.maximum(o0 + o1 + o2, 0.0)
    row = i * BN + lax.broadcasted_iota(jnp.int32, (BN, 1), 0)
    mask = (row < N).astype(jnp.float32)
    s0, s1, s2 = o0 * o0, o1 * o1, o2 * o2
    t = s1 / (s1 + s0 + 1e-07) + s2 / (s2 + s1 + 1e-07)
    part = jnp.sum(t * mask)

    @pl.when(i == 0)
    def _():
        nh_ref[0, 0] = 0.0

    nh_ref[0, 0] += part


def _k4(o0, o1, o2):
    return pl.pallas_call(
        _k4_body,
        grid=(GN,),
        in_specs=[pl.BlockSpec((BN, NOUT), lambda i: (i, 0))] * 3,
        out_specs=[
            pl.BlockSpec((BN, NOUT), lambda i: (i, 0)),
            pl.BlockSpec(memory_space=pltpu.SMEM),
        ],
        out_shape=[
            jax.ShapeDtypeStruct((NP, NOUT), jnp.float32),
            jax.ShapeDtypeStruct((1, 1), jnp.float32),
        ],
    )(o0, o1, o2)


def kernel(R, f, idx_i, idx_j, offsets, params):
    p = params
    f_pad = jnp.pad(f, ((0, NP - N), (0, 4)))

    ii_p = jnp.pad(idx_i, (0, EP - E))
    jj_p = jnp.pad(idx_j, (0, EP - E))
    rcols = R.T

    d2 = _sc_dist(rcols[0], rcols[1], rcols[2], ii_p, jj_p)
    d2 = d2.at[E:].set(4.0 * CUTOFF * CUTOFF)

    g_t = _kg(d2.reshape(GEP, BE), p)

    x = _k1(f_pad, p)

    idxi2d = ii_p.reshape(EROWS, 128)
    idxj3 = jnp.concatenate([jj_p, jj_p + NP]).reshape(2 * EROWS, 128)
    zeros_nf = jnp.zeros((NP, 32), jnp.float32)

    keys = ('Wi', 'bi', 'Wj', 'bj', 'Wri1', 'bri1', 'Wri2', 'bri2',
            'Wm', 'bm', 'u', 'Wra1', 'bra1', 'Wra2', 'bra2',
            'Wro1', 'bro1', 'Wro2', 'bro2', 'Wout', 'bout')
    pbs = [{k: p[k][b] for k in keys} for b in range(NB)]
    wijs = [jnp.concatenate([pb['Wi'], pb['Wj']], axis=1) for pb in pbs]
    bijs = [jnp.concatenate([pb['bi'], pb['bj']])[None] for pb in pbs]

    xi, xj_tab = _k2(x, wijs[0], bijs[0])
    outs = []
    for b in range(NB):
        xj_flat = xj_tab.reshape(2 * NP, 32)
        msg_flat = _SC_MSG[b](g_t, xj_flat, idxi2d, idxj3, zeros_nf)
        msg = msg_flat.reshape(2, NP, 32)

        if b + 1 < NB:
            x, out_b, xi, xj_tab = _k3(x, xi, msg, pbs[b],
                                       wijs[b + 1], bijs[b + 1])
        else:
            x, out_b = _k3(x, xi, msg, pbs[b])
        outs.append(out_b)

    outputs, nh = _k4(*outs)
    return outputs[:N], (nh[0, 0] / (N * NOUT)).reshape(())


# speedup vs baseline: 1.0061x; 1.0061x over previous
import functools
import math

import jax
import jax.numpy as jnp
from jax import lax
from jax.experimental import pallas as pl
from jax.experimental.pallas import tpu as pltpu
from jax.experimental.pallas import tpu_sc as plsc

F = 64
K = 64
CUTOFF = 10.0
NB = 3
NOUT = 2
N = 50000
E = 800000
EP = 819200
EROWS = EP // 128

BN = 3584
NP = ((N + BN - 1) // BN) * BN
GN = NP // BN
BE = 8192
GEP = EP // BE

_LOG2 = math.log(2.0)

_NSUB = 16
_NCORE = 2
_DW = EP // (2 * _NSUB)
_MROWS = EROWS // _NSUB
_SUP = 16
_NSUP = _MROWS // _SUP
_RB = 4
_GB = 2
_DCH = 6400
_NPS = NP // _NSUB


def _ssp(x):
    return jnp.maximum(x, 0.0) + jnp.log1p(jnp.exp(-jnp.abs(x))) - _LOG2


def _mm(a, w):
    return jnp.dot(a.astype(jnp.bfloat16), w.astype(jnp.bfloat16),
                   preferred_element_type=jnp.float32)


def _residual(x, W1, b1, W2, b2):
    y = _ssp(x)
    y = _ssp(_mm(y, W1) + b1)
    y = _mm(y, W2) + b2
    return x + y


def _dist_body(rx, ry, rz, ii_hbm, jj_hbm, d2_hbm, rc_v, ii_v, jj_v, d2_v):
    wid = lax.axis_index("s") * _NCORE + lax.axis_index("c")
    base = wid * _DW

    for c, r_hbm in enumerate((rx, ry, rz)):
        pltpu.sync_copy(r_hbm, rc_v)

        for t in range(_DW // _DCH):
            pltpu.sync_copy(ii_hbm.at[pl.ds(base + t * _DCH, _DCH)], ii_v)
            pltpu.sync_copy(jj_hbm.at[pl.ds(base + t * _DCH, _DCH)], jj_v)

            def body(k, _):
                sl = pl.ds(k * 16, 16)
                a = plsc.load_gather(rc_v, [ii_v[sl]])
                b = plsc.load_gather(rc_v, [jj_v[sl]])
                d = a - b
                osl = pl.ds(t * _DCH + k * 16, 16)
                if c == 0:
                    d2_v[osl] = d * d
                else:
                    d2_v[osl] += d * d
                return _

            lax.fori_loop(0, _DCH // 16, body, None)

    pltpu.sync_copy(d2_v, d2_hbm.at[pl.ds(base, _DW)])


@functools.partial(
    pl.kernel,
    out_type=jax.ShapeDtypeStruct((EP,), jnp.float32),
    mesh=plsc.VectorSubcoreMesh(core_axis_name="c", subcore_axis_name="s"),
    compiler_params=pltpu.CompilerParams(needs_layout_passes=False,
                                         use_tc_tiling_on_sc=False),
    scratch_types=[
        pltpu.VMEM((N,), jnp.float32),
        pltpu.VMEM((_DCH,), jnp.int32),
        pltpu.VMEM((_DCH,), jnp.int32),
        pltpu.VMEM((_DW,), jnp.float32),
    ],
)
def _sc_dist(rx, ry, rz, ii_hbm, jj_hbm, d2_hbm, rc_v, ii_v, jj_v, d2_v):
    _dist_body(rx, ry, rz, ii_hbm, jj_hbm, d2_hbm, rc_v, ii_v, jj_v, d2_v)


def _make_sc_msg(boff):
    def _msg_body(g_hbm, xj_hbm, ii_hbm, jj_hbm, zero_hbm, msg_hbm,
                  rows, gbuf, ii_sup, jj_sup, acc_sh, sem_in, sem_s):
        c = lax.axis_index("c")
        s = lax.axis_index("s")
        f0 = boff + c * 32

        pltpu.sync_copy(zero_hbm.at[pl.ds(s * _NPS, _NPS)],
                        acc_sh.at[pl.ds(s * _NPS, _NPS)])
        plsc.subcore_barrier()

        def fire(sup, j):
            row = s * _MROWS + sup * _SUP + j
            rb, gb = j % _RB, j % _GB
            pltpu.async_copy(xj_hbm.at[jj_sup.at[j]], rows[rb], sem_in[rb])
            pltpu.async_copy(g_hbm.at[pl.ds(row * 128, 128), pl.ds(f0, 32)],
                             gbuf[gb], sem_in[rb])

        def wait_in(j):
            rb, gb = j % _RB, j % _GB
            pltpu.make_async_copy(xj_hbm.at[jj_sup.at[j]], rows[rb],
                                  sem_in[rb]).wait()
            pltpu.make_async_copy(g_hbm.at[pl.ds(0, 128), pl.ds(f0, 32)],
                                  gbuf[gb], sem_in[rb]).wait()

        def fire_scatter(j):
            rb = j % _RB
            pltpu.async_copy(rows[rb], acc_sh.at[ii_sup.at[j]], sem_s[rb],
                             add=True)

        def drain_scatter(j):
            rb = j % _RB
            pltpu.make_async_copy(rows[rb], acc_sh.at[ii_sup.at[j]],
                                  sem_s[rb]).wait()

        def superblock(sup, _):
            @pl.when(sup > 0)
            def _():
                for j in range(_SUP - _RB, _SUP):
                    drain_scatter(j)

            r0 = s * _MROWS + sup * _SUP
            pltpu.sync_copy(ii_hbm.at[pl.ds(r0, _SUP)], ii_sup)
            pltpu.sync_copy(jj_hbm.at[pl.ds(c * EROWS + r0, _SUP)], jj_sup)

            for j in range(2):
                fire(sup, j)

            for j in range(_SUP):
                wait_in(j)

                def mul(e, _):
                    rb = j % _RB
                    gb = j % _GB
                    rows[rb][e, pl.ds(0, 16)] *= gbuf[gb][e, pl.ds(0, 16)]
                    rows[rb][e, pl.ds(16, 16)] *= gbuf[gb][e, pl.ds(16, 16)]
                    return _

                lax.fori_loop(0, 128, mul, None)
                fire_scatter(j)
                if j + 2 < _SUP:
                    if j - 2 >= 0:
                        drain_scatter(j - 2)
                    fire(sup, j + 2)
            return _

        lax.fori_loop(0, _NSUP, superblock, None)
        for j in range(_SUP - _RB, _SUP):
            drain_scatter(j)

        plsc.subcore_barrier()
        pltpu.sync_copy(acc_sh.at[pl.ds(s * _NPS, _NPS)],
                        msg_hbm.at[pl.ds(c * NP + s * _NPS, _NPS)])

    return functools.partial(
        pl.kernel,
        out_type=jax.ShapeDtypeStruct((2 * NP, 32), jnp.float32),
        mesh=plsc.VectorSubcoreMesh(core_axis_name="c", subcore_axis_name="s"),
        compiler_params=pltpu.CompilerParams(needs_layout_passes=False,
                                             use_tc_tiling_on_sc=False),
        scratch_types=[
            [pltpu.VMEM((128, 32), jnp.float32)] * _RB,
            [pltpu.VMEM((128, 32), jnp.float32)] * _GB,
            pltpu.VMEM((_SUP, 128), jnp.int32),
            pltpu.VMEM((_SUP, 128), jnp.int32),
            pltpu.VMEM_SHARED((NP, 32), jnp.float32),
            [pltpu.SemaphoreType.DMA] * _RB,
            [pltpu.SemaphoreType.DMA] * _RB,
        ],
    )(_msg_body)


_SC_MSG = [_make_sc_msg(b * F) for b in range(NB)]


def _k1_body(f_ref, w0_ref, b0_ref, w1_ref, b1_ref, w2_ref, b2_ref, x_ref):
    x = _ssp(_mm(f_ref[...], w0_ref[...]) + b0_ref[...])
    x = _ssp(_mm(x, w1_ref[...]) + b1_ref[...])
    x_ref[...] = _mm(x, w2_ref[...]) + b2_ref[...]


def _k1(f_pad, p):
    w0 = jnp.pad(p['W_em0'], ((0, 4), (0, 0)))
    return pl.pallas_call(
        _k1_body,
        grid=(GN,),
        in_specs=[
            pl.BlockSpec((BN, 8), lambda i: (i, 0)),
            pl.BlockSpec((8, F), lambda i: (0, 0)),
            pl.BlockSpec((1, F), lambda i: (0, 0)),
            pl.BlockSpec((F, F), lambda i: (0, 0)),
            pl.BlockSpec((1, F), lambda i: (0, 0)),
            pl.BlockSpec((F, F), lambda i: (0, 0)),
            pl.BlockSpec((1, F), lambda i: (0, 0)),
        ],
        out_specs=pl.BlockSpec((BN, F), lambda i: (i, 0)),
        out_shape=jax.ShapeDtypeStruct((NP, F), jnp.float32),
    )(f_pad, w0, p['b_em0'][None], p['W_em1'], p['b_em1'][None],
      p['W_em2'], p['b_em2'][None])


def _kg_body(d2_ref, cen_ref, wid_ref, wg_ref, g_ref):
    i = pl.program_id(0)
    d2 = d2_ref[pl.ds(i, 1), :]
    d = jnp.sqrt(jnp.maximum(d2, 0.0))
    xx = d * (1.0 / CUTOFF)
    x3 = xx * xx * xx
    x4 = x3 * xx
    x5 = x4 * xx
    cf = jnp.where(xx < 1.0, 1.0 - 6.0 * x5 + 15.0 * x4 - 10.0 * x3, 0.0)
    ed = jnp.broadcast_to(jnp.exp(-d), (K, BE))
    t = ed - cen_ref[...]
    rbf = jnp.broadcast_to(cf, (K, BE)) * jnp.exp(-wid_ref[...] * t * t)
    y = lax.dot_general(rbf.astype(jnp.bfloat16), wg_ref[...].astype(jnp.bfloat16),
                        dimension_numbers=(((0,), (0,)), ((), ())),
                        preferred_element_type=jnp.float32)
    g_ref[...] = jnp.concatenate([y, jnp.zeros((BE, 64), jnp.float32)], axis=1)


def _kg(d2, p):
    wg_cat = jnp.concatenate([p['Wg'][b] for b in range(NB)], axis=1)
    return pl.pallas_call(
        _kg_body,
        grid=(GEP,),
        in_specs=[
            pl.BlockSpec((GEP, BE), lambda i: (0, 0)),
            pl.BlockSpec((K, 1), lambda i: (0, 0)),
            pl.BlockSpec((K, 1), lambda i: (0, 0)),
            pl.BlockSpec((K, NB * F), lambda i: (0, 0)),
        ],
        out_specs=pl.BlockSpec((BE, 256), lambda i: (i, 0)),
        out_shape=jax.ShapeDtypeStruct((EP, 256), jnp.float32),
    )(d2, p['centers'][:, None], p['widths'][:, None], wg_cat)


def _k2_body(x_ref, wij_ref, bij_ref, xi_ref, xj_ref):
    xa = _ssp(x_ref[...])
    y = _mm(xa, wij_ref[...]) + bij_ref[...]
    xi_ref[...] = y[:, :F]
    xj_ref[0] = y[:, F:F + 32]
    xj_ref[1] = y[:, F + 32:]


def _k2(x, wij, bij):
    return pl.pallas_call(
        _k2_body,
        grid=(GN,),
        in_specs=[
            pl.BlockSpec((BN, F), lambda i: (i, 0)),
            pl.BlockSpec((F, 2 * F), lambda i: (0, 0)),
            pl.BlockSpec((1, 2 * F), lambda i: (0, 0)),
        ],
        out_specs=[
            pl.BlockSpec((BN, F), lambda i: (i, 0)),
            pl.BlockSpec((2, BN, 32), lambda i: (0, i, 0)),
        ],
        out_shape=[
            jax.ShapeDtypeStruct((NP, F), jnp.float32),
            jax.ShapeDtypeStruct((2, NP, 32), jnp.float32),
        ],
    )(x, wij, bij)


def _k3_body_fused(x_ref, xi_ref, msg_ref,
                   wri1_ref, bri1_ref, wri2_ref, bri2_ref, wm_ref, bm_ref, u_ref,
                   wra1_ref, bra1_ref, wra2_ref, bra2_ref,
                   wro1_ref, bro1_ref, wro2_ref, bro2_ref, wout_ref, bout_ref,
                   wij2_ref, bij2_ref,
                   xn_ref, out_ref, xi2_ref, xj2_ref):
    x = x_ref[...]
    m = xi_ref[...] + jnp.concatenate([msg_ref[0], msg_ref[1]], axis=1)
    for r in range(3):
        m = _residual(m, wri1_ref[r], bri1_ref[r][None], wri2_ref[r], bri2_ref[r][None])
    m = _ssp(m)
    xn = u_ref[...] * x + _mm(m, wm_ref[...]) + bm_ref[...]
    for r in range(2):
        xn = _residual(xn, wra1_ref[r], bra1_ref[r][None], wra2_ref[r], bra2_ref[r][None])
    y = _residual(xn, wro1_ref[...], bro1_ref[...], wro2_ref[...], bro2_ref[...])
    y = _ssp(y)
    out = _mm(y, wout_ref[...]) + bout_ref[...]
    xn_ref[...] = xn
    out_ref[...] = out
    if xi2_ref is not None:
        y2 = _mm(_ssp(xn), wij2_ref[...]) + bij2_ref[...]
        xi2_ref[...] = y2[:, :F]
        xj2_ref[0] = y2[:, F:F + 32]
        xj2_ref[1] = y2[:, F + 32:]


def _k3(x, xi, msg, pb, wij2=None, bij2=None):
    full = lambda shape: pl.BlockSpec(shape, lambda i: tuple(0 for _ in shape))
    fused = wij2 is not None
    in_specs = [
        pl.BlockSpec((BN, F), lambda i: (i, 0)),
        pl.BlockSpec((BN, F), lambda i: (i, 0)),
        pl.BlockSpec((2, BN, 32), lambda i: (0, i, 0)),
        full((3, F, F)), full((3, F)), full((3, F, F)), full((3, F)),
        full((F, F)), full((1, F)), full((1, F)),
        full((2, F, F)), full((2, F)), full((2, F, F)), full((2, F)),
        full((F, F)), full((1, F)), full((F, F)), full((1, F)),
        full((F, NOUT)), full((1, NOUT)),
    ]
    out_specs = [
        pl.BlockSpec((BN, F), lambda i: (i, 0)),
        pl.BlockSpec((BN, NOUT), lambda i: (i, 0)),
    ]
    out_shape = [
        jax.ShapeDtypeStruct((NP, F), jnp.float32),
        jax.ShapeDtypeStruct((NP, NOUT), jnp.float32),
    ]
    args = [x, xi, msg,
            pb['Wri1'], pb['bri1'], pb['Wri2'], pb['bri2'],
            pb['Wm'], pb['bm'][None], pb['u'][None],
            pb['Wra1'], pb['bra1'], pb['Wra2'], pb['bra2'],
            pb['Wro1'], pb['bro1'][None], pb['Wro2'], pb['bro2'][None],
            pb['Wout'], pb['bout'][None]]
    if fused:
        in_specs += [full((F, 2 * F)), full((1, 2 * F))]
        out_specs += [
            pl.BlockSpec((BN, F), lambda i: (i, 0)),
            pl.BlockSpec((2, BN, 32), lambda i: (0, i, 0)),
        ]
        out_shape += [
            jax.ShapeDtypeStruct((NP, F), jnp.float32),
            jax.ShapeDtypeStruct((2, NP, 32), jnp.float32),
        ]
        args += [wij2, bij2]
        body = _k3_body_fused
    else:
        def body(*refs):
            _k3_body_fused(*refs[:20], None, None, *refs[20:], None, None)
    return pl.pallas_call(
        body,
        grid=(GN,),
        in_specs=in_specs,
        out_specs=out_specs,
        out_shape=out_shape,
    )(*args)


def _k4_body(o0_ref, o1_ref, o2_ref, out_ref, nh_ref):
    i = pl.program_id(0)
    o0, o1, o2 = o0_ref[...], o1_ref[...], o2_ref[...]
    out_ref[...] = jnp.maximum(o0 + o1 + o2, 0.0)
    row = i * BN + lax.broadcasted_iota(jnp.int32, (BN, 1), 0)
    mask = (row < N).astype(jnp.float32)
    s0, s1, s2 = o0 * o0, o1 * o1, o2 * o2
    t = s1 / (s1 + s0 + 1e-07) + s2 / (s2 + s1 + 1e-07)
    part = jnp.sum(t * mask)

    @pl.when(i == 0)
    def _():
        nh_ref[0, 0] = 0.0

    nh_ref[0, 0] += part


def _k4(o0, o1, o2):
    return pl.pallas_call(
        _k4_body,
        grid=(GN,),
        in_specs=[pl.BlockSpec((BN, NOUT), lambda i: (i, 0))] * 3,
        out_specs=[
            pl.BlockSpec((BN, NOUT), lambda i: (i, 0)),
            pl.BlockSpec(memory_space=pltpu.SMEM),
        ],
        out_shape=[
            jax.ShapeDtypeStruct((NP, NOUT), jnp.float32),
            jax.ShapeDtypeStruct((1, 1), jnp.float32),
        ],
    )(o0, o1, o2)


def kernel(R, f, idx_i, idx_j, offsets, params):
    p = params
    f_pad = jnp.pad(f, ((0, NP - N), (0, 4)))

    ii_p = jnp.pad(idx_i, (0, EP - E))
    jj_p = jnp.pad(idx_j, (0, EP - E))
    rcols = R.T

    d2 = _sc_dist(rcols[0], rcols[1], rcols[2], ii_p, jj_p)
    d2 = d2.at[E:].set(4.0 * CUTOFF * CUTOFF)

    g_t = _kg(d2.reshape(GEP, BE), p)

    x = _k1(f_pad, p)

    idxi2d = ii_p.reshape(EROWS, 128)
    idxj3 = jnp.concatenate([jj_p, jj_p + NP]).reshape(2 * EROWS, 128)
    zeros_nf = jnp.zeros((NP, 32), jnp.float32)

    keys = ('Wi', 'bi', 'Wj', 'bj', 'Wri1', 'bri1', 'Wri2', 'bri2',
            'Wm', 'bm', 'u', 'Wra1', 'bra1', 'Wra2', 'bra2',
            'Wro1', 'bro1', 'Wro2', 'bro2', 'Wout', 'bout')
    outs = []
    for b in range(NB):
        pb = {k: p[k][b] for k in keys}
        wij = jnp.concatenate([pb['Wi'], pb['Wj']], axis=1)
        bij = jnp.concatenate([pb['bi'], pb['bj']])[None]
        xi, xj_tab = _k2(x, wij, bij)

        xj_flat = xj_tab.reshape(2 * NP, 32)
        msg_flat = _SC_MSG[b](g_t, xj_flat, idxi2d, idxj3, zeros_nf)
        msg = msg_flat.reshape(2, NP, 32)

        x, out_b = _k3(x, xi, msg, pb)
        outs.append(out_b)

    outputs, nh = _k4(*outs)
    return outputs[:N], (nh[0, 0] / (N * NOUT)).reshape(())
